# Initial kernel scaffold; baseline (speedup 1.0000x reference)
import jax
import jax.numpy as jnp
from jax.experimental import pallas as pl


def kernel(edge_index, emb_table, W1, b1, W2, b2):
    raise NotImplementedError("write your pallas kernel here")


# speedup vs baseline: 1.3746x; 1.3746x over previous
import functools

import jax
import jax.numpy as jnp
from jax import lax
from jax.experimental import pallas as pl
from jax.experimental.pallas import tpu as pltpu
from jax.experimental.pallas import tpu_sc as plsc

N = 10000
E = 320000
EMB = 128
HID = 256

NC = 2
NS = 16
NW = NC * NS
NB = 320
NPAD = NW * NB
CAP = 16384
CH = 4000
G = 256
D = EMB

_NEG_INF = float("-inf")


def _sc_agg_body(x_hbm, src_hbm, dst_hbm, h_hbm,
                 dstc_v, srcc_v, dstl_v, srcl_v, agg_v, rows_v, sem):
    wid = lax.axis_index("s") * NC + lax.axis_index("c")
    lo = wid * NB

    def prefill(i, _):
        dstl_v[pl.ds(i * 16, 16)] = jnp.full((16,), NB, jnp.int32)
        srcl_v[pl.ds(i * 16, 16)] = jnp.zeros((16,), jnp.int32)
        return 0
    lax.fori_loop(0, CAP // 16, prefill, 0)

    def init_agg(r, _):
        for f in range(D // 16):
            agg_v[r, pl.ds(f * 16, 16)] = jnp.full((16,), _NEG_INF, jnp.float32)
        return 0
    lax.fori_loop(0, NB + 1, init_agg, 0)

    def scan_chunk(c, cnt):
        pltpu.sync_copy(dst_hbm.at[pl.ds(c * CH, CH)], dstc_v)
        pltpu.sync_copy(src_hbm.at[pl.ds(c * CH, CH)], srcc_v)

        def scan16(i, cnt):
            d = dstc_v[pl.ds(i * 16, 16)]
            s = srcc_v[pl.ds(i * 16, 16)]
            dl = d - lo
            m = (dl >= 0) & (dl < NB)
            pos = cnt + plsc.cumsum(m.astype(jnp.int32)) - 1
            m2 = m & (pos < CAP)
            plsc.store_scatter(dstl_v, [pos], dl, mask=m2)
            plsc.store_scatter(srcl_v, [pos], s, mask=m2)
            su = plsc.all_reduce_population_count(m2)
            return jnp.minimum(cnt + su, CAP)
        return lax.fori_loop(0, CH // 16, scan16, cnt)

    cnt = lax.fori_loop(0, E // CH, scan_chunk, jnp.zeros((16,), jnp.int32))
    cnt_s = jnp.max(cnt, axis=0)
    nch = (cnt_s + (G - 1)) // G

    def proc(g, _):
        base = g * G
        pltpu.async_copy(x_hbm.at[srcl_v.at[pl.ds(base, G)]], rows_v, sem).wait()

        def grp(j, _):
            d_vec = dstl_v[pl.ds(base + j * 16, 16)]
            for l in range(16):
                dlo = d_vec[l]
                i = j * 16 + l
                for f in range(D // 16):
                    a = agg_v[dlo, pl.ds(f * 16, 16)]
                    r = rows_v[i, pl.ds(f * 16, 16)]
                    agg_v[dlo, pl.ds(f * 16, 16)] = jnp.maximum(a, r)
            return 0
        lax.fori_loop(0, G // 16, grp, 0)
        return 0
    lax.fori_loop(0, nch, proc, 0)

    for off, width in ((0, G), (G, NB - G)):
        pltpu.sync_copy(x_hbm.at[pl.ds(lo + off, width)], rows_v.at[pl.ds(0, width)])

        def fin(r, _):
            for f in range(D // 16):
                a = agg_v[off + r, pl.ds(f * 16, 16)]
                xx = rows_v[r, pl.ds(f * 16, 16)]
                fixed = jnp.where(a == _NEG_INF, jnp.float32(0.0), a)
                agg_v[off + r, pl.ds(f * 16, 16)] = xx + fixed
            return 0
        lax.fori_loop(0, width, fin, 0)
        pltpu.sync_copy(agg_v.at[pl.ds(off, width)], h_hbm.at[pl.ds(lo + off, width)])


def _sc_gather_max(xp, src, dst):
    mesh = plsc.VectorSubcoreMesh(core_axis_name="c", subcore_axis_name="s")
    f = pl.kernel(
        _sc_agg_body,
        out_type=jax.ShapeDtypeStruct((NPAD, D), jnp.float32),
        mesh=mesh,
        scratch_types=[
            pltpu.VMEM((CH,), jnp.int32),
            pltpu.VMEM((CH,), jnp.int32),
            pltpu.VMEM((CAP,), jnp.int32),
            pltpu.VMEM((CAP,), jnp.int32),
            pltpu.VMEM((NB + 1, D), jnp.float32),
            pltpu.VMEM((G, D), jnp.float32),
            pltpu.SemaphoreType.DMA,
        ],
        compiler_params=pltpu.CompilerParams(needs_layout_passes=False),
    )
    return f(xp, src, dst)


def _mm_body(x_ref, w_ref, b_ref, o_ref, *, relu):
    y = jnp.dot(x_ref[...], w_ref[...], preferred_element_type=jnp.float32)
    y = y + b_ref[...]
    if relu:
        y = jnp.maximum(y, 0.0)
    o_ref[...] = y


def _tc_mm(h, W, b, relu):
    M, K = h.shape
    O = W.shape[1]
    BM = 2000
    return pl.pallas_call(
        functools.partial(_mm_body, relu=relu),
        grid=(M // BM,),
        in_specs=[
            pl.BlockSpec((BM, K), lambda i: (i, 0)),
            pl.BlockSpec((K, O), lambda i: (0, 0)),
            pl.BlockSpec((1, O), lambda i: (0, 0)),
        ],
        out_specs=pl.BlockSpec((BM, O), lambda i: (i, 0)),
        out_shape=jax.ShapeDtypeStruct((M, O), jnp.float32),
    )(h, W, b.reshape(1, O))


def kernel(edge_index, emb_table, W1, b1, W2, b2):
    src = edge_index[0].astype(jnp.int32)
    dst = edge_index[1].astype(jnp.int32)
    xp = jnp.zeros((NPAD, EMB), jnp.float32).at[:N].set(emb_table)
    h1 = _sc_gather_max(xp, src, dst)
    x1 = _tc_mm(h1[:N], W1, b1, relu=True)
    x1p = jnp.zeros((NPAD, HID), jnp.float32).at[:N].set(x1)
    h2a = _sc_gather_max(x1p[:, :EMB], src, dst)
    h2b = _sc_gather_max(x1p[:, EMB:], src, dst)
    h2 = jnp.concatenate([h2a[:N], h2b[:N]], axis=1)
    return _tc_mm(h2, W2, b2, relu=False)


# speedup vs baseline: 2.4086x; 1.7522x over previous
import functools

import jax
import jax.numpy as jnp
from jax import lax
from jax.experimental import pallas as pl
from jax.experimental.pallas import tpu as pltpu
from jax.experimental.pallas import tpu_sc as plsc

N = 10000
E = 320000
EMB = 128
HID = 256

NC = 2
NS = 16
NW = NC * NS
NB = 320
NPAD = NW * NB
G = 192
CAP = 86 * G
CH = 4000
NCH = E // CH
D = EMB

_NEG_INF = float("-inf")


def _sc_bin_body(src_hbm, dst_hbm, ldst_hbm, lsrc_hbm, cnt_hbm,
                 dstc0, srcc0, dstc1, srcc1, dstl_v, srcl_v,
                 sd0, ss0, sd1, ss1):
    wid = lax.axis_index("s") * NC + lax.axis_index("c")
    lo = wid * NB

    def prefill(i, _):
        dstl_v[pl.ds(i * 16, 16)] = jnp.full((16,), NB, jnp.int32)
        srcl_v[pl.ds(i * 16, 16)] = jnp.zeros((16,), jnp.int32)
        return 0
    lax.fori_loop(0, CAP // 16, prefill, 0)

    def start(c, dstc, srcc, semd, sems):
        pltpu.async_copy(dst_hbm.at[pl.ds(c * CH, CH)], dstc, semd)
        pltpu.async_copy(src_hbm.at[pl.ds(c * CH, CH)], srcc, sems)

    def wait(dstc, srcc, semd, sems):
        pltpu.make_async_copy(dst_hbm.at[pl.ds(0, CH)], dstc, semd).wait()
        pltpu.make_async_copy(src_hbm.at[pl.ds(0, CH)], srcc, sems).wait()

    def scan_buf(dstc, srcc, cnt):
        def scan16(i, cnt):
            d = dstc[pl.ds(i * 16, 16)]
            s = srcc[pl.ds(i * 16, 16)]
            dl = d - lo
            m = (dl >= 0) & (dl < NB)
            pos = cnt + plsc.cumsum(m.astype(jnp.int32)) - 1
            m2 = m & (pos < CAP)
            plsc.store_scatter(dstl_v, [pos], dl, mask=m2)
            plsc.store_scatter(srcl_v, [pos], s, mask=m2)
            su = plsc.all_reduce_population_count(m2)
            return jnp.minimum(cnt + su, CAP)
        return lax.fori_loop(0, CH // 16, scan16, cnt)

    start(0, dstc0, srcc0, sd0, ss0)
    start(1, dstc1, srcc1, sd1, ss1)

    def pair(p, cnt):
        c0 = 2 * p
        wait(dstc0, srcc0, sd0, ss0)
        cnt = scan_buf(dstc0, srcc0, cnt)

        @pl.when(c0 + 2 < NCH)
        def _():
            start(c0 + 2, dstc0, srcc0, sd0, ss0)

        wait(dstc1, srcc1, sd1, ss1)
        cnt = scan_buf(dstc1, srcc1, cnt)

        @pl.when(c0 + 3 < NCH)
        def _():
            start(c0 + 3, dstc1, srcc1, sd1, ss1)
        return cnt

    cnt = lax.fori_loop(0, NCH // 2, pair, jnp.zeros((16,), jnp.int32))

    for k in range(8):
        dstc0[pl.ds(k * 16, 16)] = cnt
    pltpu.sync_copy(dstc0.at[pl.ds(0, 128)], cnt_hbm.at[wid])
    pltpu.sync_copy(dstl_v, ldst_hbm.at[wid])
    pltpu.sync_copy(srcl_v, lsrc_hbm.at[wid])


def _sc_bin(src, dst):
    mesh = plsc.VectorSubcoreMesh(core_axis_name="c", subcore_axis_name="s")
    f = pl.kernel(
        _sc_bin_body,
        out_type=(
            jax.ShapeDtypeStruct((NW, CAP), jnp.int32),
            jax.ShapeDtypeStruct((NW, CAP), jnp.int32),
            jax.ShapeDtypeStruct((NW, 128), jnp.int32),
        ),
        mesh=mesh,
        scratch_types=[
            pltpu.VMEM((CH,), jnp.int32),
            pltpu.VMEM((CH,), jnp.int32),
            pltpu.VMEM((CH,), jnp.int32),
            pltpu.VMEM((CH,), jnp.int32),
            pltpu.VMEM((CAP,), jnp.int32),
            pltpu.VMEM((CAP,), jnp.int32),
            pltpu.SemaphoreType.DMA,
            pltpu.SemaphoreType.DMA,
            pltpu.SemaphoreType.DMA,
            pltpu.SemaphoreType.DMA,
        ],
        compiler_params=pltpu.CompilerParams(needs_layout_passes=False),
    )
    return f(src, dst)


def _sc_agg_body(x_hbm, ldst_hbm, lsrc_hbm, cnt_hbm, agg_hbm,
                 dstl_v, srcl_v, agg_v, rows0, rows1, cvec, sem0, sem1):
    wid = lax.axis_index("s") * NC + lax.axis_index("c")
    lo = wid * NB

    pltpu.sync_copy(ldst_hbm.at[wid], dstl_v)
    pltpu.sync_copy(lsrc_hbm.at[wid], srcl_v)
    pltpu.sync_copy(cnt_hbm.at[wid], cvec)

    def init_agg(r, _):
        for f in range(D // 16):
            agg_v[r, pl.ds(f * 16, 16)] = jnp.full((16,), _NEG_INF, jnp.float32)
        return 0
    lax.fori_loop(0, NB + 1, init_agg, 0)

    cnt_s = jnp.max(cvec[pl.ds(0, 16)], axis=0)
    nch = (cnt_s + (G - 1)) // G

    def start(g, rows, sem):
        pltpu.async_copy(x_hbm.at[srcl_v.at[pl.ds(g * G, G)]], rows, sem)

    def wait(rows, sem):
        pltpu.make_async_copy(x_hbm.at[srcl_v.at[pl.ds(0, G)]], rows, sem).wait()

    def process(base, rows):
        def grp(j, _):
            d_vec = dstl_v[pl.ds(base + j * 16, 16)]
            for l in range(16):
                dlo = d_vec[l]
                i = j * 16 + l
                for f in range(D // 16):
                    a = agg_v[dlo, pl.ds(f * 16, 16)]
                    r = rows[i, pl.ds(f * 16, 16)]
                    agg_v[dlo, pl.ds(f * 16, 16)] = jnp.maximum(a, r)
            return 0
        lax.fori_loop(0, G // 16, grp, 0)

    @pl.when(nch > 0)
    def _():
        start(0, rows0, sem0)

    @pl.when(nch > 1)
    def _():
        start(1, rows1, sem1)

    def pair(p, _):
        g0 = 2 * p

        @pl.when(g0 < nch)
        def _():
            wait(rows0, sem0)
            process(g0 * G, rows0)

            @pl.when(g0 + 2 < nch)
            def _():
                start(g0 + 2, rows0, sem0)

        @pl.when(g0 + 1 < nch)
        def _():
            wait(rows1, sem1)
            process((g0 + 1) * G, rows1)

            @pl.when(g0 + 3 < nch)
            def _():
                start(g0 + 3, rows1, sem1)
        return 0

    lax.fori_loop(0, (nch + 1) // 2, pair, 0)

    pltpu.sync_copy(agg_v.at[pl.ds(0, NB)], agg_hbm.at[pl.ds(lo, NB)])


def _sc_agg(xp, ldst, lsrc, cnts):
    mesh = plsc.VectorSubcoreMesh(core_axis_name="c", subcore_axis_name="s")
    f = pl.kernel(
        _sc_agg_body,
        out_type=jax.ShapeDtypeStruct((NPAD, D), jnp.float32),
        mesh=mesh,
        scratch_types=[
            pltpu.VMEM((CAP,), jnp.int32),
            pltpu.VMEM((CAP,), jnp.int32),
            pltpu.VMEM((NB + 1, D), jnp.float32),
            pltpu.VMEM((G, D), jnp.float32),
            pltpu.VMEM((G, D), jnp.float32),
            pltpu.VMEM((128,), jnp.int32),
            pltpu.SemaphoreType.DMA,
            pltpu.SemaphoreType.DMA,
        ],
        compiler_params=pltpu.CompilerParams(needs_layout_passes=False),
    )
    return f(xp, ldst, lsrc, cnts)


def _mm_body(x_ref, a_ref, w_ref, b_ref, o_ref, *, relu):
    a = a_ref[...]
    h = x_ref[...] + jnp.where(a == _NEG_INF, 0.0, a)
    y = jnp.dot(h, w_ref[...], preferred_element_type=jnp.float32) + b_ref[...]
    if relu:
        y = jnp.maximum(y, 0.0)
    o_ref[...] = y


def _tc_mm(x, agg, W, b, relu):
    M, K = x.shape
    O = W.shape[1]
    BM = 2000
    return pl.pallas_call(
        functools.partial(_mm_body, relu=relu),
        grid=(M // BM,),
        in_specs=[
            pl.BlockSpec((BM, K), lambda i: (i, 0)),
            pl.BlockSpec((BM, K), lambda i: (i, 0)),
            pl.BlockSpec((K, O), lambda i: (0, 0)),
            pl.BlockSpec((1, O), lambda i: (0, 0)),
        ],
        out_specs=pl.BlockSpec((BM, O), lambda i: (i, 0)),
        out_shape=jax.ShapeDtypeStruct((M, O), jnp.float32),
    )(x, agg, W, b.reshape(1, O))


def kernel(edge_index, emb_table, W1, b1, W2, b2):
    src = edge_index[0].astype(jnp.int32)
    dst = edge_index[1].astype(jnp.int32)
    ldst, lsrc, cnts = _sc_bin(src, dst)
    xp = jnp.zeros((NPAD, EMB), jnp.float32).at[:N].set(emb_table)
    agg1 = _sc_agg(xp, ldst, lsrc, cnts)
    x1 = _tc_mm(emb_table, agg1[:N], W1, b1, relu=True)
    x1p = jnp.zeros((NPAD, HID), jnp.float32).at[:N].set(x1)
    agg2a = _sc_agg(x1p[:, :EMB], ldst, lsrc, cnts)
    agg2b = _sc_agg(x1p[:, EMB:], ldst, lsrc, cnts)
    agg2 = jnp.concatenate([agg2a[:N], agg2b[:N]], axis=1)
    return _tc_mm(x1, agg2, W2, b2, relu=False)


# speedup vs baseline: 3.4298x; 1.4240x over previous
import functools

import jax
import jax.numpy as jnp
from jax import lax
from jax.experimental import pallas as pl
from jax.experimental.pallas import tpu as pltpu
from jax.experimental.pallas import tpu_sc as plsc

N = 10000
E = 320000
EMB = 128
HID = 256

NC = 2
NS = 16
NW = NC * NS
NB = 320
NPAD = NW * NB
G = 192
CAP = 86 * G
CH = 4000
NCH = E // CH
D = EMB

_NEG_INF = float("-inf")


def _sc_bin_body(src_hbm, dst_hbm, ldst_hbm, lsrc_hbm, cnt_hbm,
                 dstc0, srcc0, dstc1, srcc1, dstl_v, srcl_v,
                 sdst_v, ssrc_v, hist_sm,
                 sd0, ss0, sd1, ss1):
    wid = lax.axis_index("s") * NC + lax.axis_index("c")
    lo = wid * NB

    def prefill(i, _):
        dstl_v[pl.ds(i * 16, 16)] = jnp.full((16,), NB, jnp.int32)
        srcl_v[pl.ds(i * 16, 16)] = jnp.zeros((16,), jnp.int32)
        sdst_v[pl.ds(i * 16, 16)] = jnp.full((16,), NB, jnp.int32)
        ssrc_v[pl.ds(i * 16, 16)] = jnp.zeros((16,), jnp.int32)
        return 0
    lax.fori_loop(0, CAP // 16, prefill, 0)

    def start(c, dstc, srcc, semd, sems):
        pltpu.async_copy(dst_hbm.at[pl.ds(c * CH, CH)], dstc, semd)
        pltpu.async_copy(src_hbm.at[pl.ds(c * CH, CH)], srcc, sems)

    def wait(dstc, srcc, semd, sems):
        pltpu.make_async_copy(dst_hbm.at[pl.ds(0, CH)], dstc, semd).wait()
        pltpu.make_async_copy(src_hbm.at[pl.ds(0, CH)], srcc, sems).wait()

    def scan_buf(dstc, srcc, cnt):
        def scan16(i, cnt):
            d = dstc[pl.ds(i * 16, 16)]
            s = srcc[pl.ds(i * 16, 16)]
            dl = d - lo
            m = (dl >= 0) & (dl < NB)
            pos = cnt + plsc.cumsum(m.astype(jnp.int32)) - 1
            m2 = m & (pos < CAP)
            plsc.store_scatter(dstl_v, [pos], dl, mask=m2)
            plsc.store_scatter(srcl_v, [pos], s, mask=m2)
            su = plsc.all_reduce_population_count(m2)
            return jnp.minimum(cnt + su, CAP)
        return lax.fori_loop(0, CH // 16, scan16, cnt)

    start(0, dstc0, srcc0, sd0, ss0)
    start(1, dstc1, srcc1, sd1, ss1)

    def pair(p, cnt):
        c0 = 2 * p
        wait(dstc0, srcc0, sd0, ss0)
        cnt = scan_buf(dstc0, srcc0, cnt)

        @pl.when(c0 + 2 < NCH)
        def _():
            start(c0 + 2, dstc0, srcc0, sd0, ss0)

        wait(dstc1, srcc1, sd1, ss1)
        cnt = scan_buf(dstc1, srcc1, cnt)

        @pl.when(c0 + 3 < NCH)
        def _():
            start(c0 + 3, dstc1, srcc1, sd1, ss1)
        return cnt

    cnt = lax.fori_loop(0, NCH // 2, pair, jnp.zeros((16,), jnp.int32))
    cnt_s = jnp.max(cnt, axis=0)
    ngrp = (cnt_s + 15) // 16

    def zero_hist(r, _):
        hist_sm[r] = 0
        return 0
    lax.fori_loop(0, NB + 1, zero_hist, 0)

    def hist_grp(j, _):
        d_vec = dstl_v[pl.ds(j * 16, 16)]
        for l in range(16):
            dlo = d_vec[l]
            hist_sm[dlo] = hist_sm[dlo] + 1
        return 0
    lax.fori_loop(0, ngrp, hist_grp, 0)

    def prefix(r, run):
        v = hist_sm[r]
        hist_sm[r] = run
        return run + v
    lax.fori_loop(0, NB + 1, prefix, jnp.int32(0))

    lane_iota = lax.iota(jnp.int32, 16)

    def place_grp(j, _):
        d_vec = dstl_v[pl.ds(j * 16, 16)]
        s_vec = srcl_v[pl.ds(j * 16, 16)]
        posv = jnp.zeros((16,), jnp.int32)
        for l in range(16):
            dlo = d_vec[l]
            p = hist_sm[dlo]
            hist_sm[dlo] = p + 1
            posv = jnp.where(lane_iota == l, p, posv)
        plsc.store_scatter(sdst_v, [posv], d_vec)
        plsc.store_scatter(ssrc_v, [posv], s_vec)
        return 0
    lax.fori_loop(0, ngrp, place_grp, 0)

    for k in range(8):
        dstc0[pl.ds(k * 16, 16)] = cnt
    pltpu.sync_copy(dstc0.at[pl.ds(0, 128)], cnt_hbm.at[wid])
    pltpu.sync_copy(sdst_v, ldst_hbm.at[wid])
    pltpu.sync_copy(ssrc_v, lsrc_hbm.at[wid])


def _sc_bin(src, dst):
    mesh = plsc.VectorSubcoreMesh(core_axis_name="c", subcore_axis_name="s")
    f = pl.kernel(
        _sc_bin_body,
        out_type=(
            jax.ShapeDtypeStruct((NW, CAP), jnp.int32),
            jax.ShapeDtypeStruct((NW, CAP), jnp.int32),
            jax.ShapeDtypeStruct((NW, 128), jnp.int32),
        ),
        mesh=mesh,
        scratch_types=[
            pltpu.VMEM((CH,), jnp.int32),
            pltpu.VMEM((CH,), jnp.int32),
            pltpu.VMEM((CH,), jnp.int32),
            pltpu.VMEM((CH,), jnp.int32),
            pltpu.VMEM((CAP,), jnp.int32),
            pltpu.VMEM((CAP,), jnp.int32),
            pltpu.VMEM((CAP,), jnp.int32),
            pltpu.VMEM((CAP,), jnp.int32),
            pltpu.SMEM((NB + 8,), jnp.int32),
            pltpu.SemaphoreType.DMA,
            pltpu.SemaphoreType.DMA,
            pltpu.SemaphoreType.DMA,
            pltpu.SemaphoreType.DMA,
        ],
        compiler_params=pltpu.CompilerParams(needs_layout_passes=False),
    )
    return f(src, dst)


def _sc_agg_body(x_hbm, ldst_hbm, lsrc_hbm, cnt_hbm, agg_hbm,
                 dstl_v, srcl_v, agg_v, rows0, rows1, cvec, sem0, sem1):
    wid = lax.axis_index("s") * NC + lax.axis_index("c")
    lo = wid * NB

    pltpu.sync_copy(ldst_hbm.at[wid], dstl_v)
    pltpu.sync_copy(lsrc_hbm.at[wid], srcl_v)
    pltpu.sync_copy(cnt_hbm.at[wid], cvec)

    def init_agg(r, _):
        for f in range(D // 16):
            agg_v[r, pl.ds(f * 16, 16)] = jnp.full((16,), _NEG_INF, jnp.float32)
        return 0
    lax.fori_loop(0, NB + 1, init_agg, 0)

    cnt_s = jnp.max(cvec[pl.ds(0, 16)], axis=0)
    nch = (cnt_s + (G - 1)) // G

    def start(g, rows, sem):
        pltpu.async_copy(x_hbm.at[srcl_v.at[pl.ds(g * G, G)]], rows, sem)

    def wait(rows, sem):
        pltpu.make_async_copy(x_hbm.at[srcl_v.at[pl.ds(0, G)]], rows, sem).wait()

    neg = jnp.full((16,), _NEG_INF, jnp.float32)

    def process(base, rows):
        def grp(j, carry):
            prev = carry[0]
            acc = list(carry[1:])
            d_vec = dstl_v[pl.ds(base + j * 16, 16)]
            for l in range(16):
                dlo = d_vec[l]
                i = j * 16 + l
                row = [rows[i, pl.ds(f * 16, 16)] for f in range(D // 16)]
                pred = dlo == prev

                @pl.when(jnp.logical_not(pred))
                def _(prev=prev, acc=acc):
                    for f in range(D // 16):
                        a = agg_v[prev, pl.ds(f * 16, 16)]
                        agg_v[prev, pl.ds(f * 16, 16)] = jnp.maximum(a, acc[f])

                acc = [jnp.where(pred, jnp.maximum(acc[f], row[f]), row[f])
                       for f in range(D // 16)]
                prev = dlo
            return (prev, *acc)

        carry = lax.fori_loop(0, G // 16, grp,
                              (jnp.int32(NB), *([neg] * (D // 16))))
        prev = carry[0]
        acc = carry[1:]
        for f in range(D // 16):
            a = agg_v[prev, pl.ds(f * 16, 16)]
            agg_v[prev, pl.ds(f * 16, 16)] = jnp.maximum(a, acc[f])

    @pl.when(nch > 0)
    def _():
        start(0, rows0, sem0)

    @pl.when(nch > 1)
    def _():
        start(1, rows1, sem1)

    def pair(p, _):
        g0 = 2 * p

        @pl.when(g0 < nch)
        def _():
            wait(rows0, sem0)
            process(g0 * G, rows0)

            @pl.when(g0 + 2 < nch)
            def _():
                start(g0 + 2, rows0, sem0)

        @pl.when(g0 + 1 < nch)
        def _():
            wait(rows1, sem1)
            process((g0 + 1) * G, rows1)

            @pl.when(g0 + 3 < nch)
            def _():
                start(g0 + 3, rows1, sem1)
        return 0

    lax.fori_loop(0, (nch + 1) // 2, pair, 0)

    pltpu.sync_copy(agg_v.at[pl.ds(0, NB)], agg_hbm.at[pl.ds(lo, NB)])


def _sc_agg(xp, ldst, lsrc, cnts):
    mesh = plsc.VectorSubcoreMesh(core_axis_name="c", subcore_axis_name="s")
    f = pl.kernel(
        _sc_agg_body,
        out_type=jax.ShapeDtypeStruct((NPAD, D), jnp.float32),
        mesh=mesh,
        scratch_types=[
            pltpu.VMEM((CAP,), jnp.int32),
            pltpu.VMEM((CAP,), jnp.int32),
            pltpu.VMEM((NB + 1, D), jnp.float32),
            pltpu.VMEM((G, D), jnp.float32),
            pltpu.VMEM((G, D), jnp.float32),
            pltpu.VMEM((128,), jnp.int32),
            pltpu.SemaphoreType.DMA,
            pltpu.SemaphoreType.DMA,
        ],
        compiler_params=pltpu.CompilerParams(needs_layout_passes=False),
    )
    return f(xp, ldst, lsrc, cnts)


def _mm_body(x_ref, a_ref, w_ref, b_ref, o_ref, *, relu):
    a = a_ref[...]
    h = x_ref[...] + jnp.where(a == _NEG_INF, 0.0, a)
    y = jnp.dot(h, w_ref[...], preferred_element_type=jnp.float32) + b_ref[...]
    if relu:
        y = jnp.maximum(y, 0.0)
    o_ref[...] = y


def _tc_mm(x, agg, W, b, relu):
    M, K = x.shape
    O = W.shape[1]
    BM = 2000
    return pl.pallas_call(
        functools.partial(_mm_body, relu=relu),
        grid=(M // BM,),
        in_specs=[
            pl.BlockSpec((BM, K), lambda i: (i, 0)),
            pl.BlockSpec((BM, K), lambda i: (i, 0)),
            pl.BlockSpec((K, O), lambda i: (0, 0)),
            pl.BlockSpec((1, O), lambda i: (0, 0)),
        ],
        out_specs=pl.BlockSpec((BM, O), lambda i: (i, 0)),
        out_shape=jax.ShapeDtypeStruct((M, O), jnp.float32),
    )(x, agg, W, b.reshape(1, O))


def kernel(edge_index, emb_table, W1, b1, W2, b2):
    src = edge_index[0].astype(jnp.int32)
    dst = edge_index[1].astype(jnp.int32)
    ldst, lsrc, cnts = _sc_bin(src, dst)
    xp = jnp.zeros((NPAD, EMB), jnp.float32).at[:N].set(emb_table)
    agg1 = _sc_agg(xp, ldst, lsrc, cnts)
    x1 = _tc_mm(emb_table, agg1[:N], W1, b1, relu=True)
    x1p = jnp.zeros((NPAD, HID), jnp.float32).at[:N].set(x1)
    agg2a = _sc_agg(x1p[:, :EMB], ldst, lsrc, cnts)
    agg2b = _sc_agg(x1p[:, EMB:], ldst, lsrc, cnts)
    agg2 = jnp.concatenate([agg2a[:N], agg2b[:N]], axis=1)
    return _tc_mm(x1, agg2, W2, b2, relu=False)


# speedup vs baseline: 3.5894x; 1.0465x over previous
import functools

import jax
import jax.numpy as jnp
from jax import lax
from jax.experimental import pallas as pl
from jax.experimental.pallas import tpu as pltpu
from jax.experimental.pallas import tpu_sc as plsc

N = 10000
E = 320000
EMB = 128
HID = 256

NC = 2
NS = 16
NW = NC * NS
NB = 320
NPAD = NW * NB
G = 192
CAP = 86 * G
CH = 4000
NCH = E // CH
D = EMB

_NEG_INF = float("-inf")


def _sc_bin_body(src_hbm, dst_hbm, ldst_hbm, lsrc_hbm, cnt_hbm,
                 dstc0, srcc0, dstc1, srcc1, dstl_v, srcl_v,
                 sdst_v, ssrc_v, hist_sm,
                 sd0, ss0, sd1, ss1):
    wid = lax.axis_index("s") * NC + lax.axis_index("c")
    lo = wid * NB

    def prefill(i, _):
        dstl_v[pl.ds(i * 16, 16)] = jnp.full((16,), NB, jnp.int32)
        srcl_v[pl.ds(i * 16, 16)] = jnp.zeros((16,), jnp.int32)
        sdst_v[pl.ds(i * 16, 16)] = jnp.full((16,), NB, jnp.int32)
        ssrc_v[pl.ds(i * 16, 16)] = jnp.zeros((16,), jnp.int32)
        return 0
    lax.fori_loop(0, CAP // 16, prefill, 0)

    def start(c, dstc, srcc, semd, sems):
        pltpu.async_copy(dst_hbm.at[pl.ds(c * CH, CH)], dstc, semd)
        pltpu.async_copy(src_hbm.at[pl.ds(c * CH, CH)], srcc, sems)

    def wait(dstc, srcc, semd, sems):
        pltpu.make_async_copy(dst_hbm.at[pl.ds(0, CH)], dstc, semd).wait()
        pltpu.make_async_copy(src_hbm.at[pl.ds(0, CH)], srcc, sems).wait()

    def scan_buf(dstc, srcc, cnt):
        def scan16(off, cnt):
            d = dstc[pl.ds(off, 16)]
            s = srcc[pl.ds(off, 16)]
            dl = d - lo
            m = (dl >= 0) & (dl < NB)
            pos = cnt + plsc.cumsum(m.astype(jnp.int32)) - 1
            m2 = m & (pos < CAP)
            plsc.store_scatter(dstl_v, [pos], dl, mask=m2)
            plsc.store_scatter(srcl_v, [pos], s, mask=m2)
            su = plsc.all_reduce_population_count(m2)
            return jnp.minimum(cnt + su, CAP)

        def scan32(i, cnt):
            cnt = scan16(i * 32, cnt)
            return scan16(i * 32 + 16, cnt)
        return lax.fori_loop(0, CH // 32, scan32, cnt)

    start(0, dstc0, srcc0, sd0, ss0)
    start(1, dstc1, srcc1, sd1, ss1)

    def pair(p, cnt):
        c0 = 2 * p
        wait(dstc0, srcc0, sd0, ss0)
        cnt = scan_buf(dstc0, srcc0, cnt)

        @pl.when(c0 + 2 < NCH)
        def _():
            start(c0 + 2, dstc0, srcc0, sd0, ss0)

        wait(dstc1, srcc1, sd1, ss1)
        cnt = scan_buf(dstc1, srcc1, cnt)

        @pl.when(c0 + 3 < NCH)
        def _():
            start(c0 + 3, dstc1, srcc1, sd1, ss1)
        return cnt

    cnt = lax.fori_loop(0, NCH // 2, pair, jnp.zeros((16,), jnp.int32))
    cnt_s = jnp.max(cnt, axis=0)
    ngrp = (cnt_s + 15) // 16

    def zero_hist(r, _):
        hist_sm[r] = 0
        return 0
    lax.fori_loop(0, NB + 1, zero_hist, 0)

    def hist_grp(j, _):
        d_vec = dstl_v[pl.ds(j * 16, 16)]
        for l in range(16):
            dlo = d_vec[l]
            hist_sm[dlo] = hist_sm[dlo] + 1
        return 0
    lax.fori_loop(0, ngrp, hist_grp, 0)

    def prefix(r, run):
        v = hist_sm[r]
        hist_sm[r] = run
        return run + v
    lax.fori_loop(0, NB + 1, prefix, jnp.int32(0))

    lane_iota = lax.iota(jnp.int32, 16)

    def place_grp(j, _):
        d_vec = dstl_v[pl.ds(j * 16, 16)]
        s_vec = srcl_v[pl.ds(j * 16, 16)]
        posv = jnp.zeros((16,), jnp.int32)
        for l in range(16):
            dlo = d_vec[l]
            p = hist_sm[dlo]
            hist_sm[dlo] = p + 1
            posv = jnp.where(lane_iota == l, p, posv)
        plsc.store_scatter(sdst_v, [posv], d_vec)
        plsc.store_scatter(ssrc_v, [posv], s_vec)
        return 0
    lax.fori_loop(0, ngrp, place_grp, 0)

    for k in range(8):
        dstc0[pl.ds(k * 16, 16)] = cnt
    pltpu.sync_copy(dstc0.at[pl.ds(0, 128)], cnt_hbm.at[wid])
    pltpu.sync_copy(sdst_v, ldst_hbm.at[wid])
    pltpu.sync_copy(ssrc_v, lsrc_hbm.at[wid])


def _sc_bin(src, dst):
    mesh = plsc.VectorSubcoreMesh(core_axis_name="c", subcore_axis_name="s")
    f = pl.kernel(
        _sc_bin_body,
        out_type=(
            jax.ShapeDtypeStruct((NW, CAP), jnp.int32),
            jax.ShapeDtypeStruct((NW, CAP), jnp.int32),
            jax.ShapeDtypeStruct((NW, 128), jnp.int32),
        ),
        mesh=mesh,
        scratch_types=[
            pltpu.VMEM((CH,), jnp.int32),
            pltpu.VMEM((CH,), jnp.int32),
            pltpu.VMEM((CH,), jnp.int32),
            pltpu.VMEM((CH,), jnp.int32),
            pltpu.VMEM((CAP,), jnp.int32),
            pltpu.VMEM((CAP,), jnp.int32),
            pltpu.VMEM((CAP,), jnp.int32),
            pltpu.VMEM((CAP,), jnp.int32),
            pltpu.SMEM((NB + 8,), jnp.int32),
            pltpu.SemaphoreType.DMA,
            pltpu.SemaphoreType.DMA,
            pltpu.SemaphoreType.DMA,
            pltpu.SemaphoreType.DMA,
        ],
        compiler_params=pltpu.CompilerParams(needs_layout_passes=False),
    )
    return f(src, dst)


def _sc_agg_body(x_hbm, ldst_hbm, lsrc_hbm, cnt_hbm, agg_hbm,
                 dstl_v, srcl_v, agg_v, rows0, rows1, cvec, sem0, sem1):
    wid = lax.axis_index("s") * NC + lax.axis_index("c")
    lo = wid * NB

    pltpu.sync_copy(ldst_hbm.at[wid], dstl_v)
    pltpu.sync_copy(lsrc_hbm.at[wid], srcl_v)
    pltpu.sync_copy(cnt_hbm.at[wid], cvec)

    def init_agg(r, _):
        for f in range(D // 16):
            agg_v[r, pl.ds(f * 16, 16)] = jnp.full((16,), _NEG_INF, jnp.float32)
        return 0
    lax.fori_loop(0, NB + 1, init_agg, 0)

    cnt_s = jnp.max(cvec[pl.ds(0, 16)], axis=0)
    nch = (cnt_s + (G - 1)) // G

    def start(g, rows, sem):
        pltpu.async_copy(x_hbm.at[srcl_v.at[pl.ds(g * G, G)]], rows, sem)

    def wait(rows, sem):
        pltpu.make_async_copy(x_hbm.at[srcl_v.at[pl.ds(0, G)]], rows, sem).wait()

    neg = jnp.full((16,), _NEG_INF, jnp.float32)
    lane_iota = lax.iota(jnp.int32, 16)

    def process(base, rows):
        def grp(j, carry):
            d_vec = dstl_v[pl.ds(base + j * 16, 16)]
            idxm1 = jnp.maximum(base + j * 16 - 1 + lane_iota, 0)
            d_sh = plsc.load_gather(dstl_v, [idxm1])
            anyb = (jnp.max(d_vec, axis=0) != jnp.min(d_vec, axis=0)) | (d_vec[0] != d_sh[0])

            def fast(carry):
                prev = carry[0]
                acc = list(carry[1:])
                for f in range(D // 16):
                    m = acc[f]
                    for l in range(16):
                        m = jnp.maximum(m, rows[j * 16 + l, pl.ds(f * 16, 16)])
                    acc[f] = m
                return (d_vec[15], *acc)

            def slow(carry):
                prev = carry[0]
                acc = list(carry[1:])
                for l in range(16):
                    dlo = d_vec[l]
                    i = j * 16 + l
                    row = [rows[i, pl.ds(f * 16, 16)] for f in range(D // 16)]
                    pred = dlo == prev

                    @pl.when(jnp.logical_not(pred))
                    def _(prev=prev, acc=acc):
                        for f in range(D // 16):
                            a = agg_v[prev, pl.ds(f * 16, 16)]
                            agg_v[prev, pl.ds(f * 16, 16)] = jnp.maximum(a, acc[f])

                    acc = [jnp.where(pred, jnp.maximum(acc[f], row[f]), row[f])
                           for f in range(D // 16)]
                    prev = dlo
                return (prev, *acc)

            return lax.cond(anyb, slow, fast, carry)

        carry = lax.fori_loop(0, G // 16, grp,
                              (jnp.int32(NB), *([neg] * (D // 16))))
        prev = carry[0]
        acc = carry[1:]
        for f in range(D // 16):
            a = agg_v[prev, pl.ds(f * 16, 16)]
            agg_v[prev, pl.ds(f * 16, 16)] = jnp.maximum(a, acc[f])

    @pl.when(nch > 0)
    def _():
        start(0, rows0, sem0)

    @pl.when(nch > 1)
    def _():
        start(1, rows1, sem1)

    def pair(p, _):
        g0 = 2 * p

        @pl.when(g0 < nch)
        def _():
            wait(rows0, sem0)
            process(g0 * G, rows0)

            @pl.when(g0 + 2 < nch)
            def _():
                start(g0 + 2, rows0, sem0)

        @pl.when(g0 + 1 < nch)
        def _():
            wait(rows1, sem1)
            process((g0 + 1) * G, rows1)

            @pl.when(g0 + 3 < nch)
            def _():
                start(g0 + 3, rows1, sem1)
        return 0

    lax.fori_loop(0, (nch + 1) // 2, pair, 0)

    pltpu.sync_copy(agg_v.at[pl.ds(0, NB)], agg_hbm.at[pl.ds(lo, NB)])


def _sc_agg(xp, ldst, lsrc, cnts):
    mesh = plsc.VectorSubcoreMesh(core_axis_name="c", subcore_axis_name="s")
    f = pl.kernel(
        _sc_agg_body,
        out_type=jax.ShapeDtypeStruct((NPAD, D), jnp.float32),
        mesh=mesh,
        scratch_types=[
            pltpu.VMEM((CAP,), jnp.int32),
            pltpu.VMEM((CAP,), jnp.int32),
            pltpu.VMEM((NB + 1, D), jnp.float32),
            pltpu.VMEM((G, D), jnp.float32),
            pltpu.VMEM((G, D), jnp.float32),
            pltpu.VMEM((128,), jnp.int32),
            pltpu.SemaphoreType.DMA,
            pltpu.SemaphoreType.DMA,
        ],
        compiler_params=pltpu.CompilerParams(needs_layout_passes=False),
    )
    return f(xp, ldst, lsrc, cnts)


def _mm_body(x_ref, a_ref, w_ref, b_ref, o_ref, *, relu):
    a = a_ref[...]
    h = x_ref[...] + jnp.where(a == _NEG_INF, 0.0, a)
    y = jnp.dot(h, w_ref[...], preferred_element_type=jnp.float32) + b_ref[...]
    if relu:
        y = jnp.maximum(y, 0.0)
    o_ref[...] = y


def _tc_mm(x, agg, W, b, relu):
    M, K = x.shape
    O = W.shape[1]
    BM = 2000
    return pl.pallas_call(
        functools.partial(_mm_body, relu=relu),
        grid=(M // BM,),
        in_specs=[
            pl.BlockSpec((BM, K), lambda i: (i, 0)),
            pl.BlockSpec((BM, K), lambda i: (i, 0)),
            pl.BlockSpec((K, O), lambda i: (0, 0)),
            pl.BlockSpec((1, O), lambda i: (0, 0)),
        ],
        out_specs=pl.BlockSpec((BM, O), lambda i: (i, 0)),
        out_shape=jax.ShapeDtypeStruct((M, O), jnp.float32),
    )(x, agg, W, b.reshape(1, O))


def kernel(edge_index, emb_table, W1, b1, W2, b2):
    src = edge_index[0].astype(jnp.int32)
    dst = edge_index[1].astype(jnp.int32)
    ldst, lsrc, cnts = _sc_bin(src, dst)
    xp = jnp.zeros((NPAD, EMB), jnp.float32).at[:N].set(emb_table)
    agg1 = _sc_agg(xp, ldst, lsrc, cnts)
    x1 = _tc_mm(emb_table, agg1[:N], W1, b1, relu=True)
    x1p = jnp.zeros((NPAD, HID), jnp.float32).at[:N].set(x1)
    agg2a = _sc_agg(x1p[:, :EMB], ldst, lsrc, cnts)
    agg2b = _sc_agg(x1p[:, EMB:], ldst, lsrc, cnts)
    agg2 = jnp.concatenate([agg2a[:N], agg2b[:N]], axis=1)
    return _tc_mm(x1, agg2, W2, b2, relu=False)


# speedup vs baseline: 4.4139x; 1.2297x over previous
import functools

import jax
import jax.numpy as jnp
from jax import lax
from jax.experimental import pallas as pl
from jax.experimental.pallas import tpu as pltpu
from jax.experimental.pallas import tpu_sc as plsc

N = 10000
E = 320000
EMB = 128
HID = 256

NC = 2
NS = 16
NW = NC * NS
E2 = E // 2
NB = 640
NPAD = NS * NB
G = 64
CAP = 195 * G
CH = 4000
NCH = E2 // CH
D = EMB

_NEG_INF = float("-inf")


def _sc_bin_body(src_hbm, dst_hbm, ldst_hbm, lsrc_hbm, cnt_hbm,
                 dstc0, srcc0, dstc1, srcc1, dstl_v, srcl_v,
                 sdst_v, ssrc_v, hist_sm,
                 sd0, ss0, sd1, ss1):
    sc = lax.axis_index("c")
    t = lax.axis_index("s")
    wid = t * NC + sc
    ebase = sc * E2
    lo = t * NB

    def prefill(i, _):
        dstl_v[pl.ds(i * 16, 16)] = jnp.full((16,), NB, jnp.int32)
        srcl_v[pl.ds(i * 16, 16)] = jnp.zeros((16,), jnp.int32)
        sdst_v[pl.ds(i * 16, 16)] = jnp.full((16,), NB, jnp.int32)
        ssrc_v[pl.ds(i * 16, 16)] = jnp.zeros((16,), jnp.int32)
        return 0
    lax.fori_loop(0, CAP // 16, prefill, 0)

    def start(c, dstc, srcc, semd, sems):
        pltpu.async_copy(dst_hbm.at[pl.ds(ebase + c * CH, CH)], dstc, semd)
        pltpu.async_copy(src_hbm.at[pl.ds(ebase + c * CH, CH)], srcc, sems)

    def wait(dstc, srcc, semd, sems):
        pltpu.make_async_copy(dst_hbm.at[pl.ds(0, CH)], dstc, semd).wait()
        pltpu.make_async_copy(src_hbm.at[pl.ds(0, CH)], srcc, sems).wait()

    def scan_buf(dstc, srcc, cnt):
        def scan16(off, cnt):
            d = dstc[pl.ds(off, 16)]
            s = srcc[pl.ds(off, 16)]
            dl = d - lo
            m = (dl >= 0) & (dl < NB)
            pos = cnt + plsc.cumsum(m.astype(jnp.int32)) - 1
            m2 = m & (pos < CAP)
            plsc.store_scatter(dstl_v, [pos], dl, mask=m2)
            plsc.store_scatter(srcl_v, [pos], s, mask=m2)
            su = plsc.all_reduce_population_count(m2)
            return jnp.minimum(cnt + su, CAP)

        def scan32(i, cnt):
            cnt = scan16(i * 32, cnt)
            return scan16(i * 32 + 16, cnt)
        return lax.fori_loop(0, CH // 32, scan32, cnt)

    start(0, dstc0, srcc0, sd0, ss0)
    start(1, dstc1, srcc1, sd1, ss1)

    def pair(p, cnt):
        c0 = 2 * p
        wait(dstc0, srcc0, sd0, ss0)
        cnt = scan_buf(dstc0, srcc0, cnt)

        @pl.when(c0 + 2 < NCH)
        def _():
            start(c0 + 2, dstc0, srcc0, sd0, ss0)

        wait(dstc1, srcc1, sd1, ss1)
        cnt = scan_buf(dstc1, srcc1, cnt)

        @pl.when(c0 + 3 < NCH)
        def _():
            start(c0 + 3, dstc1, srcc1, sd1, ss1)
        return cnt

    cnt = lax.fori_loop(0, NCH // 2, pair, jnp.zeros((16,), jnp.int32))
    cnt_s = jnp.max(cnt, axis=0)
    ngrp = (cnt_s + 15) // 16

    def zero_hist(r, _):
        hist_sm[r] = 0
        return 0
    lax.fori_loop(0, NB + 1, zero_hist, 0)

    def hist_grp(j, _):
        d_vec = dstl_v[pl.ds(j * 16, 16)]
        for l in range(16):
            dlo = d_vec[l]
            hist_sm[dlo] = hist_sm[dlo] + 1
        return 0
    lax.fori_loop(0, ngrp, hist_grp, 0)

    def prefix(r, run):
        v = hist_sm[r]
        hist_sm[r] = run
        return run + v
    lax.fori_loop(0, NB + 1, prefix, jnp.int32(0))

    lane_iota = lax.iota(jnp.int32, 16)

    def place_grp(j, _):
        d_vec = dstl_v[pl.ds(j * 16, 16)]
        s_vec = srcl_v[pl.ds(j * 16, 16)]
        posv = jnp.zeros((16,), jnp.int32)
        for l in range(16):
            dlo = d_vec[l]
            p = hist_sm[dlo]
            hist_sm[dlo] = p + 1
            posv = jnp.where(lane_iota == l, p, posv)
        plsc.store_scatter(sdst_v, [posv], d_vec)
        plsc.store_scatter(ssrc_v, [posv], s_vec)
        return 0
    lax.fori_loop(0, ngrp, place_grp, 0)

    for k in range(8):
        dstc0[pl.ds(k * 16, 16)] = cnt
    pltpu.sync_copy(dstc0.at[pl.ds(0, 128)], cnt_hbm.at[wid])
    pltpu.sync_copy(sdst_v, ldst_hbm.at[wid])
    pltpu.sync_copy(ssrc_v, lsrc_hbm.at[wid])


def _sc_bin(src, dst):
    mesh = plsc.VectorSubcoreMesh(core_axis_name="c", subcore_axis_name="s")
    f = pl.kernel(
        _sc_bin_body,
        out_type=(
            jax.ShapeDtypeStruct((NW, CAP), jnp.int32),
            jax.ShapeDtypeStruct((NW, CAP), jnp.int32),
            jax.ShapeDtypeStruct((NW, 128), jnp.int32),
        ),
        mesh=mesh,
        scratch_types=[
            pltpu.VMEM((CH,), jnp.int32),
            pltpu.VMEM((CH,), jnp.int32),
            pltpu.VMEM((CH,), jnp.int32),
            pltpu.VMEM((CH,), jnp.int32),
            pltpu.VMEM((CAP,), jnp.int32),
            pltpu.VMEM((CAP,), jnp.int32),
            pltpu.VMEM((CAP,), jnp.int32),
            pltpu.VMEM((CAP,), jnp.int32),
            pltpu.SMEM((NB + 8,), jnp.int32),
            pltpu.SemaphoreType.DMA,
            pltpu.SemaphoreType.DMA,
            pltpu.SemaphoreType.DMA,
            pltpu.SemaphoreType.DMA,
        ],
        compiler_params=pltpu.CompilerParams(needs_layout_passes=False),
    )
    return f(src, dst)


def _sc_agg_body(x_hbm, ldst_hbm, lsrc_hbm, cnt_hbm, agg_hbm,
                 dstl_v, srcl_v, agg_v, rows0, rows1, cvec, sem0, sem1):
    sc = lax.axis_index("c")
    t = lax.axis_index("s")
    wid = t * NC + sc
    lo = sc * NPAD + t * NB

    pltpu.sync_copy(ldst_hbm.at[wid], dstl_v)
    pltpu.sync_copy(lsrc_hbm.at[wid], srcl_v)
    pltpu.sync_copy(cnt_hbm.at[wid], cvec)

    def init_agg(r, _):
        for f in range(D // 16):
            agg_v[r, pl.ds(f * 16, 16)] = jnp.full((16,), _NEG_INF, jnp.float32)
        return 0
    lax.fori_loop(0, NB + 1, init_agg, 0)

    cnt_s = jnp.max(cvec[pl.ds(0, 16)], axis=0)
    nch = (cnt_s + (G - 1)) // G

    def start(g, rows, sem):
        pltpu.async_copy(x_hbm.at[srcl_v.at[pl.ds(g * G, G)]], rows, sem)

    def wait(rows, sem):
        pltpu.make_async_copy(x_hbm.at[srcl_v.at[pl.ds(0, G)]], rows, sem).wait()

    neg = jnp.full((16,), _NEG_INF, jnp.float32)
    lane_iota = lax.iota(jnp.int32, 16)

    def process(base, rows):
        def grp(j, carry):
            d_vec = dstl_v[pl.ds(base + j * 16, 16)]
            idxm1 = jnp.maximum(base + j * 16 - 1 + lane_iota, 0)
            d_sh = plsc.load_gather(dstl_v, [idxm1])
            anyb = (jnp.max(d_vec, axis=0) != jnp.min(d_vec, axis=0)) | (d_vec[0] != d_sh[0])

            def fast(carry):
                prev = carry[0]
                acc = list(carry[1:])
                for f in range(D // 16):
                    m = acc[f]
                    for l in range(16):
                        m = jnp.maximum(m, rows[j * 16 + l, pl.ds(f * 16, 16)])
                    acc[f] = m
                return (d_vec[15], *acc)

            def slow(carry):
                prev = carry[0]
                acc = list(carry[1:])
                for l in range(16):
                    dlo = d_vec[l]
                    i = j * 16 + l
                    row = [rows[i, pl.ds(f * 16, 16)] for f in range(D // 16)]
                    pred = dlo == prev

                    @pl.when(jnp.logical_not(pred))
                    def _(prev=prev, acc=acc):
                        for f in range(D // 16):
                            a = agg_v[prev, pl.ds(f * 16, 16)]
                            agg_v[prev, pl.ds(f * 16, 16)] = jnp.maximum(a, acc[f])

                    acc = [jnp.where(pred, jnp.maximum(acc[f], row[f]), row[f])
                           for f in range(D // 16)]
                    prev = dlo
                return (prev, *acc)

            return lax.cond(anyb, slow, fast, carry)

        carry = lax.fori_loop(0, G // 16, grp,
                              (jnp.int32(NB), *([neg] * (D // 16))))
        prev = carry[0]
        acc = carry[1:]
        for f in range(D // 16):
            a = agg_v[prev, pl.ds(f * 16, 16)]
            agg_v[prev, pl.ds(f * 16, 16)] = jnp.maximum(a, acc[f])

    @pl.when(nch > 0)
    def _():
        start(0, rows0, sem0)

    @pl.when(nch > 1)
    def _():
        start(1, rows1, sem1)

    def pair(p, _):
        g0 = 2 * p

        @pl.when(g0 < nch)
        def _():
            wait(rows0, sem0)
            process(g0 * G, rows0)

            @pl.when(g0 + 2 < nch)
            def _():
                start(g0 + 2, rows0, sem0)

        @pl.when(g0 + 1 < nch)
        def _():
            wait(rows1, sem1)
            process((g0 + 1) * G, rows1)

            @pl.when(g0 + 3 < nch)
            def _():
                start(g0 + 3, rows1, sem1)
        return 0

    lax.fori_loop(0, (nch + 1) // 2, pair, 0)

    pltpu.sync_copy(agg_v.at[pl.ds(0, NB)], agg_hbm.at[pl.ds(lo, NB)])


def _sc_agg(xp, ldst, lsrc, cnts):
    mesh = plsc.VectorSubcoreMesh(core_axis_name="c", subcore_axis_name="s")
    f = pl.kernel(
        _sc_agg_body,
        out_type=jax.ShapeDtypeStruct((2 * NPAD, D), jnp.float32),
        mesh=mesh,
        scratch_types=[
            pltpu.VMEM((CAP,), jnp.int32),
            pltpu.VMEM((CAP,), jnp.int32),
            pltpu.VMEM((NB + 1, D), jnp.float32),
            pltpu.VMEM((G, D), jnp.float32),
            pltpu.VMEM((G, D), jnp.float32),
            pltpu.VMEM((128,), jnp.int32),
            pltpu.SemaphoreType.DMA,
            pltpu.SemaphoreType.DMA,
        ],
        compiler_params=pltpu.CompilerParams(needs_layout_passes=False),
    )
    return f(xp, ldst, lsrc, cnts)


def _mm_body(x_ref, aa_ref, ab_ref, w_ref, b_ref, o_ref, *, relu):
    a = jnp.maximum(aa_ref[...], ab_ref[...])
    h = x_ref[...] + jnp.where(a == _NEG_INF, 0.0, a)
    y = jnp.dot(h, w_ref[...], preferred_element_type=jnp.float32) + b_ref[...]
    if relu:
        y = jnp.maximum(y, 0.0)
    o_ref[...] = y


def _tc_mm(x, aggA, aggB, W, b, relu):
    M, K = x.shape
    O = W.shape[1]
    BM = 2000
    return pl.pallas_call(
        functools.partial(_mm_body, relu=relu),
        grid=(M // BM,),
        in_specs=[
            pl.BlockSpec((BM, K), lambda i: (i, 0)),
            pl.BlockSpec((BM, K), lambda i: (i, 0)),
            pl.BlockSpec((BM, K), lambda i: (i, 0)),
            pl.BlockSpec((K, O), lambda i: (0, 0)),
            pl.BlockSpec((1, O), lambda i: (0, 0)),
        ],
        out_specs=pl.BlockSpec((BM, O), lambda i: (i, 0)),
        out_shape=jax.ShapeDtypeStruct((M, O), jnp.float32),
    )(x, aggA, aggB, W, b.reshape(1, O))


def kernel(edge_index, emb_table, W1, b1, W2, b2):
    src = edge_index[0].astype(jnp.int32)
    dst = edge_index[1].astype(jnp.int32)
    ldst, lsrc, cnts = _sc_bin(src, dst)
    xp = jnp.zeros((NPAD, EMB), jnp.float32).at[:N].set(emb_table)
    agg1 = _sc_agg(xp, ldst, lsrc, cnts)
    x1 = _tc_mm(emb_table, agg1[:N], agg1[NPAD:NPAD + N], W1, b1, relu=True)
    x1p = jnp.zeros((NPAD, HID), jnp.float32).at[:N].set(x1)
    agg2a = _sc_agg(x1p[:, :EMB], ldst, lsrc, cnts)
    agg2b = _sc_agg(x1p[:, EMB:], ldst, lsrc, cnts)
    agg2A = jnp.concatenate([agg2a[:N], agg2b[:N]], axis=1)
    agg2B = jnp.concatenate(
        [agg2a[NPAD:NPAD + N], agg2b[NPAD:NPAD + N]], axis=1)
    return _tc_mm(x1, agg2A, agg2B, W2, b2, relu=False)


# speedup vs baseline: 4.4997x; 1.0195x over previous
import functools

import jax
import jax.numpy as jnp
from jax import lax
from jax.experimental import pallas as pl
from jax.experimental.pallas import tpu as pltpu
from jax.experimental.pallas import tpu_sc as plsc

N = 10000
E = 320000
EMB = 128
HID = 256

NC = 2
NS = 16
NW = NC * NS
E2 = E // 2
NB = 640
NPAD = NS * NB
G = 64
CAP = 195 * G
CH = 4000
NCH = E2 // CH
HW = 656
D = EMB

_NEG_INF = float("-inf")


def _sc_bin_body(src_hbm, dst_hbm, ldst_hbm, lsrc_hbm, cnt_hbm,
                 dstc0, srcc0, dstc1, srcc1, dstl_v, srcl_v,
                 sdst_v, ssrc_v, hist_v,
                 sd0, ss0, sd1, ss1):
    sc = lax.axis_index("c")
    t = lax.axis_index("s")
    wid = t * NC + sc
    ebase = sc * E2
    lo = t * NB

    def prefill(i, _):
        dstl_v[pl.ds(i * 16, 16)] = jnp.full((16,), NB, jnp.int32)
        srcl_v[pl.ds(i * 16, 16)] = jnp.zeros((16,), jnp.int32)
        sdst_v[pl.ds(i * 16, 16)] = jnp.full((16,), NB, jnp.int32)
        ssrc_v[pl.ds(i * 16, 16)] = jnp.zeros((16,), jnp.int32)
        return 0
    lax.fori_loop(0, CAP // 16, prefill, 0)

    def zero_hist(i, _):
        hist_v[pl.ds(i * 16, 16)] = jnp.zeros((16,), jnp.int32)
        return 0
    lax.fori_loop(0, HW // 16, zero_hist, 0)

    def start(c, dstc, srcc, semd, sems):
        pltpu.async_copy(dst_hbm.at[pl.ds(ebase + c * CH, CH)], dstc, semd)
        pltpu.async_copy(src_hbm.at[pl.ds(ebase + c * CH, CH)], srcc, sems)

    def wait(dstc, srcc, semd, sems):
        pltpu.make_async_copy(dst_hbm.at[pl.ds(0, CH)], dstc, semd).wait()
        pltpu.make_async_copy(src_hbm.at[pl.ds(0, CH)], srcc, sems).wait()

    def scan_buf(dstc, srcc, cnt):
        def scan16(off, cnt):
            d = dstc[pl.ds(off, 16)]
            s = srcc[pl.ds(off, 16)]
            dl = d - lo
            m = (dl >= 0) & (dl < NB)
            pos = cnt + plsc.cumsum(m.astype(jnp.int32)) - 1
            m2 = m & (pos < CAP)
            plsc.store_scatter(dstl_v, [pos], dl, mask=m2)
            plsc.store_scatter(srcl_v, [pos], s, mask=m2)
            occ, lastm = plsc.scan_count(dl, mask=m2)
            plsc.addupdate_scatter(hist_v, [dl], occ, mask=lastm)
            su = plsc.all_reduce_population_count(m2)
            return jnp.minimum(cnt + su, CAP)

        def scan32(i, cnt):
            cnt = scan16(i * 32, cnt)
            return scan16(i * 32 + 16, cnt)
        return lax.fori_loop(0, CH // 32, scan32, cnt)

    start(0, dstc0, srcc0, sd0, ss0)
    start(1, dstc1, srcc1, sd1, ss1)

    def pair(p, cnt):
        c0 = 2 * p
        wait(dstc0, srcc0, sd0, ss0)
        cnt = scan_buf(dstc0, srcc0, cnt)

        @pl.when(c0 + 2 < NCH)
        def _():
            start(c0 + 2, dstc0, srcc0, sd0, ss0)

        wait(dstc1, srcc1, sd1, ss1)
        cnt = scan_buf(dstc1, srcc1, cnt)

        @pl.when(c0 + 3 < NCH)
        def _():
            start(c0 + 3, dstc1, srcc1, sd1, ss1)
        return cnt

    cnt = lax.fori_loop(0, NCH // 2, pair, jnp.zeros((16,), jnp.int32))
    cnt_s = jnp.max(cnt, axis=0)
    ngrp = (cnt_s + 15) // 16

    def prefix(i, run):
        h = hist_v[pl.ds(i * 16, 16)]
        c = plsc.cumsum(h)
        hist_v[pl.ds(i * 16, 16)] = run + (c - h)
        return run + c[15]
    lax.fori_loop(0, HW // 16, prefix, jnp.int32(0))

    def place_grp(j, _):
        d_vec = dstl_v[pl.ds(j * 16, 16)]
        s_vec = srcl_v[pl.ds(j * 16, 16)]
        occ, lastm = plsc.scan_count(d_vec)
        base = plsc.load_gather(hist_v, [d_vec])
        posv = base + occ - 1
        plsc.store_scatter(sdst_v, [posv], d_vec)
        plsc.store_scatter(ssrc_v, [posv], s_vec)
        plsc.addupdate_scatter(hist_v, [d_vec], occ, mask=lastm)
        return 0
    lax.fori_loop(0, ngrp, place_grp, 0)

    for k in range(8):
        dstc0[pl.ds(k * 16, 16)] = cnt
    pltpu.sync_copy(dstc0.at[pl.ds(0, 128)], cnt_hbm.at[wid])
    pltpu.sync_copy(sdst_v, ldst_hbm.at[wid])
    pltpu.sync_copy(ssrc_v, lsrc_hbm.at[wid])


def _sc_bin(src, dst):
    mesh = plsc.VectorSubcoreMesh(core_axis_name="c", subcore_axis_name="s")
    f = pl.kernel(
        _sc_bin_body,
        out_type=(
            jax.ShapeDtypeStruct((NW, CAP), jnp.int32),
            jax.ShapeDtypeStruct((NW, CAP), jnp.int32),
            jax.ShapeDtypeStruct((NW, 128), jnp.int32),
        ),
        mesh=mesh,
        scratch_types=[
            pltpu.VMEM((CH,), jnp.int32),
            pltpu.VMEM((CH,), jnp.int32),
            pltpu.VMEM((CH,), jnp.int32),
            pltpu.VMEM((CH,), jnp.int32),
            pltpu.VMEM((CAP,), jnp.int32),
            pltpu.VMEM((CAP,), jnp.int32),
            pltpu.VMEM((CAP,), jnp.int32),
            pltpu.VMEM((CAP,), jnp.int32),
            pltpu.VMEM((HW,), jnp.int32),
            pltpu.SemaphoreType.DMA,
            pltpu.SemaphoreType.DMA,
            pltpu.SemaphoreType.DMA,
            pltpu.SemaphoreType.DMA,
        ],
        compiler_params=pltpu.CompilerParams(needs_layout_passes=False),
    )
    return f(src, dst)


def _sc_agg_body(x_hbm, ldst_hbm, lsrc_hbm, cnt_hbm, agg_hbm,
                 dstl_v, srcl_v, agg_v, rows0, rows1, cvec, sem0, sem1):
    sc = lax.axis_index("c")
    t = lax.axis_index("s")
    wid = t * NC + sc
    lo = t * NB

    pltpu.sync_copy(ldst_hbm.at[wid], dstl_v)
    pltpu.sync_copy(lsrc_hbm.at[wid], srcl_v)
    pltpu.sync_copy(cnt_hbm.at[wid], cvec)

    def init_agg(r, _):
        for f in range(D // 16):
            agg_v[r, pl.ds(f * 16, 16)] = jnp.full((16,), _NEG_INF, jnp.float32)
        return 0
    lax.fori_loop(0, NB + 1, init_agg, 0)

    cnt_s = jnp.max(cvec[pl.ds(0, 16)], axis=0)
    nch = (cnt_s + (G - 1)) // G

    def start(g, rows, sem):
        pltpu.async_copy(x_hbm.at[srcl_v.at[pl.ds(g * G, G)]], rows, sem)

    def wait(rows, sem):
        pltpu.make_async_copy(x_hbm.at[srcl_v.at[pl.ds(0, G)]], rows, sem).wait()

    neg = jnp.full((16,), _NEG_INF, jnp.float32)
    lane_iota = lax.iota(jnp.int32, 16)

    def process(base, rows):
        def grp(j, carry):
            d_vec = dstl_v[pl.ds(base + j * 16, 16)]
            idxm1 = jnp.maximum(base + j * 16 - 1 + lane_iota, 0)
            d_sh = plsc.load_gather(dstl_v, [idxm1])
            anyb = (jnp.max(d_vec, axis=0) != jnp.min(d_vec, axis=0)) | (d_vec[0] != d_sh[0])

            def fast(carry):
                prev = carry[0]
                acc = list(carry[1:])
                for f in range(D // 16):
                    m = acc[f]
                    for l in range(16):
                        m = jnp.maximum(m, rows[j * 16 + l, pl.ds(f * 16, 16)])
                    acc[f] = m
                return (d_vec[15], *acc)

            def slow(carry):
                prev = carry[0]
                acc = list(carry[1:])
                for l in range(16):
                    dlo = d_vec[l]
                    i = j * 16 + l
                    row = [rows[i, pl.ds(f * 16, 16)] for f in range(D // 16)]
                    pred = dlo == prev

                    @pl.when(jnp.logical_not(pred))
                    def _(prev=prev, acc=acc):
                        for f in range(D // 16):
                            a = agg_v[prev, pl.ds(f * 16, 16)]
                            agg_v[prev, pl.ds(f * 16, 16)] = jnp.maximum(a, acc[f])

                    acc = [jnp.where(pred, jnp.maximum(acc[f], row[f]), row[f])
                           for f in range(D // 16)]
                    prev = dlo
                return (prev, *acc)

            return lax.cond(anyb, slow, fast, carry)

        carry = lax.fori_loop(0, G // 16, grp,
                              (jnp.int32(NB), *([neg] * (D // 16))))
        prev = carry[0]
        acc = carry[1:]
        for f in range(D // 16):
            a = agg_v[prev, pl.ds(f * 16, 16)]
            agg_v[prev, pl.ds(f * 16, 16)] = jnp.maximum(a, acc[f])

    @pl.when(nch > 0)
    def _():
        start(0, rows0, sem0)

    @pl.when(nch > 1)
    def _():
        start(1, rows1, sem1)

    def pair(p, _):
        g0 = 2 * p

        @pl.when(g0 < nch)
        def _():
            wait(rows0, sem0)
            process(g0 * G, rows0)

            @pl.when(g0 + 2 < nch)
            def _():
                start(g0 + 2, rows0, sem0)

        @pl.when(g0 + 1 < nch)
        def _():
            wait(rows1, sem1)
            process((g0 + 1) * G, rows1)

            @pl.when(g0 + 3 < nch)
            def _():
                start(g0 + 3, rows1, sem1)
        return 0

    lax.fori_loop(0, (nch + 1) // 2, pair, 0)

    pltpu.sync_copy(agg_v.at[pl.ds(0, NB)], agg_hbm.at[sc, pl.ds(lo, NB)])


def _sc_agg(xp, ldst, lsrc, cnts):
    mesh = plsc.VectorSubcoreMesh(core_axis_name="c", subcore_axis_name="s")
    f = pl.kernel(
        _sc_agg_body,
        out_type=jax.ShapeDtypeStruct((2, NPAD, D), jnp.float32),
        mesh=mesh,
        scratch_types=[
            pltpu.VMEM((CAP,), jnp.int32),
            pltpu.VMEM((CAP,), jnp.int32),
            pltpu.VMEM((NB + 1, D), jnp.float32),
            pltpu.VMEM((G, D), jnp.float32),
            pltpu.VMEM((G, D), jnp.float32),
            pltpu.VMEM((128,), jnp.int32),
            pltpu.SemaphoreType.DMA,
            pltpu.SemaphoreType.DMA,
        ],
        compiler_params=pltpu.CompilerParams(needs_layout_passes=False),
    )
    return f(xp, ldst, lsrc, cnts)


_BM = 2000


def _agg_specs():
    return [
        pl.BlockSpec((1, _BM, EMB), lambda i: (0, i, 0)),
        pl.BlockSpec((1, _BM, EMB), lambda i: (1, i, 0)),
    ]


def _combine(x, aA, aB):
    a = jnp.maximum(aA[0], aB[0])
    return x + jnp.where(a == _NEG_INF, 0.0, a)


def _mm1_body(x_ref, aa_ref, ab_ref, w_ref, b_ref, oa_ref, ob_ref):
    h = _combine(x_ref[...], aa_ref[...], ab_ref[...])
    y = jnp.dot(h, w_ref[...], preferred_element_type=jnp.float32) + b_ref[...]
    y = jnp.maximum(y, 0.0)
    oa_ref[...] = y[:, :EMB]
    ob_ref[...] = y[:, EMB:]


def _tc_mm1(x, agg1, W1, b1):
    return pl.pallas_call(
        _mm1_body,
        grid=(N // _BM,),
        in_specs=[
            pl.BlockSpec((_BM, EMB), lambda i: (i, 0)),
            *_agg_specs(),
            pl.BlockSpec((EMB, HID), lambda i: (0, 0)),
            pl.BlockSpec((1, HID), lambda i: (0, 0)),
        ],
        out_specs=[
            pl.BlockSpec((_BM, EMB), lambda i: (i, 0)),
            pl.BlockSpec((_BM, EMB), lambda i: (i, 0)),
        ],
        out_shape=[
            jax.ShapeDtypeStruct((N, EMB), jnp.float32),
            jax.ShapeDtypeStruct((N, EMB), jnp.float32),
        ],
    )(x, agg1, agg1, W1, b1.reshape(1, HID))


def _mm2_body(xa_ref, xb_ref, a2aa_ref, a2ab_ref, a2ba_ref, a2bb_ref,
              wa_ref, wb_ref, b_ref, o_ref):
    hL = _combine(xa_ref[...], a2aa_ref[...], a2ab_ref[...])
    hR = _combine(xb_ref[...], a2ba_ref[...], a2bb_ref[...])
    y = jnp.dot(hL, wa_ref[...], preferred_element_type=jnp.float32)
    y += jnp.dot(hR, wb_ref[...], preferred_element_type=jnp.float32)
    o_ref[...] = y + b_ref[...]


def _tc_mm2(x1a, x1b, agg2a, agg2b, W2, b2):
    return pl.pallas_call(
        _mm2_body,
        grid=(N // _BM,),
        in_specs=[
            pl.BlockSpec((_BM, EMB), lambda i: (i, 0)),
            pl.BlockSpec((_BM, EMB), lambda i: (i, 0)),
            *_agg_specs(),
            *_agg_specs(),
            pl.BlockSpec((EMB, EMB), lambda i: (0, 0)),
            pl.BlockSpec((EMB, EMB), lambda i: (0, 0)),
            pl.BlockSpec((1, EMB), lambda i: (0, 0)),
        ],
        out_specs=pl.BlockSpec((_BM, EMB), lambda i: (i, 0)),
        out_shape=jax.ShapeDtypeStruct((N, EMB), jnp.float32),
    )(x1a, x1b, agg2a, agg2a, agg2b, agg2b, W2[:EMB], W2[EMB:], b2.reshape(1, EMB))


def kernel(edge_index, emb_table, W1, b1, W2, b2):
    src = edge_index[0].astype(jnp.int32)
    dst = edge_index[1].astype(jnp.int32)
    ldst, lsrc, cnts = _sc_bin(src, dst)
    agg1 = _sc_agg(emb_table, ldst, lsrc, cnts)
    x1a, x1b = _tc_mm1(emb_table, agg1, W1, b1)
    agg2a = _sc_agg(x1a, ldst, lsrc, cnts)
    agg2b = _sc_agg(x1b, ldst, lsrc, cnts)
    return _tc_mm2(x1a, x1b, agg2a, agg2b, W2, b2)


# speedup vs baseline: 5.1626x; 1.1473x over previous
import functools

import jax
import jax.numpy as jnp
from jax import lax
from jax.experimental import pallas as pl
from jax.experimental.pallas import tpu as pltpu
from jax.experimental.pallas import tpu_sc as plsc

N = 10000
E = 320000
EMB = 128
HID = 256

NC = 2
NS = 16
NW = NC * NS
E2 = E // 2
NB = 640
NPAD = NS * NB
G = 64
REG = 832
CAP = 16 * REG
CH = 4000
NCH = E2 // CH
HW = 656
D = EMB

_NEG_INF = float("-inf")


def _sc_bin_body(src_hbm, dst_hbm, ldst_hbm, lsrc_hbm, cnt_hbm,
                 dstc0, srcc0, dstc1, srcc1, dstl_v, srcl_v,
                 sdst_v, ssrc_v, hist_v,
                 sd0, ss0, sd1, ss1):
    sc = lax.axis_index("c")
    t = lax.axis_index("s")
    wid = t * NC + sc
    ebase = sc * E2
    lo = t * NB

    def prefill(i, _):
        dstl_v[pl.ds(i * 16, 16)] = jnp.full((16,), NB, jnp.int32)
        srcl_v[pl.ds(i * 16, 16)] = jnp.zeros((16,), jnp.int32)
        sdst_v[pl.ds(i * 16, 16)] = jnp.full((16,), NB, jnp.int32)
        ssrc_v[pl.ds(i * 16, 16)] = jnp.zeros((16,), jnp.int32)
        return 0
    lax.fori_loop(0, CAP // 16, prefill, 0)

    def zero_hist(i, _):
        hist_v[pl.ds(i * 16, 16)] = jnp.zeros((16,), jnp.int32)
        return 0
    lax.fori_loop(0, HW // 16, zero_hist, 0)

    def start(c, dstc, srcc, semd, sems):
        pltpu.async_copy(dst_hbm.at[pl.ds(ebase + c * CH, CH)], dstc, semd)
        pltpu.async_copy(src_hbm.at[pl.ds(ebase + c * CH, CH)], srcc, sems)

    def wait(dstc, srcc, semd, sems):
        pltpu.make_async_copy(dst_hbm.at[pl.ds(0, CH)], dstc, semd).wait()
        pltpu.make_async_copy(src_hbm.at[pl.ds(0, CH)], srcc, sems).wait()

    lane_base = lax.iota(jnp.int32, 16) * REG

    def scan_buf(dstc, srcc, cnts):
        def scan16(off, cnts):
            d = dstc[pl.ds(off, 16)]
            s = srcc[pl.ds(off, 16)]
            dl = d - lo
            m = (plsc.bitcast(dl, jnp.uint32) < jnp.uint32(NB)) & (cnts < REG)
            pos = lane_base + cnts
            plsc.store_scatter(dstl_v, [pos], dl, mask=m)
            plsc.store_scatter(srcl_v, [pos], s, mask=m)
            return cnts + m.astype(jnp.int32)

        def scan32(i, cnts):
            cnts = scan16(i * 32, cnts)
            return scan16(i * 32 + 16, cnts)
        return lax.fori_loop(0, CH // 32, scan32, cnts)

    start(0, dstc0, srcc0, sd0, ss0)
    start(1, dstc1, srcc1, sd1, ss1)

    def pair(p, cnt):
        c0 = 2 * p
        wait(dstc0, srcc0, sd0, ss0)
        cnt = scan_buf(dstc0, srcc0, cnt)

        @pl.when(c0 + 2 < NCH)
        def _():
            start(c0 + 2, dstc0, srcc0, sd0, ss0)

        wait(dstc1, srcc1, sd1, ss1)
        cnt = scan_buf(dstc1, srcc1, cnt)

        @pl.when(c0 + 3 < NCH)
        def _():
            start(c0 + 3, dstc1, srcc1, sd1, ss1)
        return cnt

    cnts = lax.fori_loop(0, NCH // 2, pair, jnp.zeros((16,), jnp.int32))
    cnt_s = jnp.sum(cnts, axis=0)

    def hist_grp(j, _):
        d_vec = dstl_v[pl.ds(j * 16, 16)]
        occ, lastm = plsc.scan_count(d_vec)
        plsc.addupdate_scatter(hist_v, [d_vec], occ, mask=lastm)
        return 0
    lax.fori_loop(0, CAP // 16, hist_grp, 0)

    def prefix(i, run):
        h = hist_v[pl.ds(i * 16, 16)]
        c = plsc.cumsum(h)
        hist_v[pl.ds(i * 16, 16)] = run + (c - h)
        return run + c[15]
    lax.fori_loop(0, HW // 16, prefix, jnp.int32(0))

    def place_grp(j, _):
        d_vec = dstl_v[pl.ds(j * 16, 16)]
        s_vec = srcl_v[pl.ds(j * 16, 16)]
        occ, lastm = plsc.scan_count(d_vec)
        base = plsc.load_gather(hist_v, [d_vec])
        posv = base + occ - 1
        plsc.store_scatter(sdst_v, [posv], d_vec)
        plsc.store_scatter(ssrc_v, [posv], s_vec)
        plsc.addupdate_scatter(hist_v, [d_vec], occ, mask=lastm)
        return 0
    lax.fori_loop(0, CAP // 16, place_grp, 0)

    cnt_spl = jnp.full((16,), cnt_s, jnp.int32)
    for k in range(8):
        dstc0[pl.ds(k * 16, 16)] = cnt_spl
    pltpu.sync_copy(dstc0.at[pl.ds(0, 128)], cnt_hbm.at[wid])
    pltpu.sync_copy(sdst_v, ldst_hbm.at[wid])
    pltpu.sync_copy(ssrc_v, lsrc_hbm.at[wid])


def _sc_bin(src, dst):
    mesh = plsc.VectorSubcoreMesh(core_axis_name="c", subcore_axis_name="s")
    f = pl.kernel(
        _sc_bin_body,
        out_type=(
            jax.ShapeDtypeStruct((NW, CAP), jnp.int32),
            jax.ShapeDtypeStruct((NW, CAP), jnp.int32),
            jax.ShapeDtypeStruct((NW, 128), jnp.int32),
        ),
        mesh=mesh,
        scratch_types=[
            pltpu.VMEM((CH,), jnp.int32),
            pltpu.VMEM((CH,), jnp.int32),
            pltpu.VMEM((CH,), jnp.int32),
            pltpu.VMEM((CH,), jnp.int32),
            pltpu.VMEM((CAP,), jnp.int32),
            pltpu.VMEM((CAP,), jnp.int32),
            pltpu.VMEM((CAP,), jnp.int32),
            pltpu.VMEM((CAP,), jnp.int32),
            pltpu.VMEM((HW,), jnp.int32),
            pltpu.SemaphoreType.DMA,
            pltpu.SemaphoreType.DMA,
            pltpu.SemaphoreType.DMA,
            pltpu.SemaphoreType.DMA,
        ],
        compiler_params=pltpu.CompilerParams(needs_layout_passes=False),
    )
    return f(src, dst)


def _sc_agg_body(x_hbm, ldst_hbm, lsrc_hbm, cnt_hbm, agg_hbm,
                 dstl_v, srcl_v, agg_v, rows0, rows1, cvec, sem0, sem1):
    sc = lax.axis_index("c")
    t = lax.axis_index("s")
    wid = t * NC + sc
    lo = t * NB

    pltpu.sync_copy(ldst_hbm.at[wid], dstl_v)
    pltpu.sync_copy(lsrc_hbm.at[wid], srcl_v)
    pltpu.sync_copy(cnt_hbm.at[wid], cvec)

    def init_agg(r, _):
        for f in range(D // 16):
            agg_v[r, pl.ds(f * 16, 16)] = jnp.full((16,), _NEG_INF, jnp.float32)
        return 0
    lax.fori_loop(0, NB + 1, init_agg, 0)

    cnt_s = jnp.max(cvec[pl.ds(0, 16)], axis=0)
    nch = (cnt_s + (G - 1)) // G

    def start(g, rows, sem):
        pltpu.async_copy(x_hbm.at[srcl_v.at[pl.ds(g * G, G)]], rows, sem)

    def wait(rows, sem):
        pltpu.make_async_copy(x_hbm.at[srcl_v.at[pl.ds(0, G)]], rows, sem).wait()

    neg = jnp.full((16,), _NEG_INF, jnp.float32)
    lane_iota = lax.iota(jnp.int32, 16)

    def process(base, rows):
        def grp(j, carry):
            d_vec = dstl_v[pl.ds(base + j * 16, 16)]
            idxm1 = jnp.maximum(base + j * 16 - 1 + lane_iota, 0)
            d_sh = plsc.load_gather(dstl_v, [idxm1])
            anyb = (jnp.max(d_vec, axis=0) != jnp.min(d_vec, axis=0)) | (d_vec[0] != d_sh[0])

            def fast(carry):
                prev = carry[0]
                acc = list(carry[1:])
                for f in range(D // 16):
                    m = acc[f]
                    for l in range(16):
                        m = jnp.maximum(m, rows[j * 16 + l, pl.ds(f * 16, 16)])
                    acc[f] = m
                return (d_vec[15], *acc)

            def slow(carry):
                prev = carry[0]
                acc = list(carry[1:])
                for l in range(16):
                    dlo = d_vec[l]
                    i = j * 16 + l
                    row = [rows[i, pl.ds(f * 16, 16)] for f in range(D // 16)]
                    pred = dlo == prev

                    @pl.when(jnp.logical_not(pred))
                    def _(prev=prev, acc=acc):
                        for f in range(D // 16):
                            a = agg_v[prev, pl.ds(f * 16, 16)]
                            agg_v[prev, pl.ds(f * 16, 16)] = jnp.maximum(a, acc[f])

                    acc = [jnp.where(pred, jnp.maximum(acc[f], row[f]), row[f])
                           for f in range(D // 16)]
                    prev = dlo
                return (prev, *acc)

            return lax.cond(anyb, slow, fast, carry)

        carry = lax.fori_loop(0, G // 16, grp,
                              (jnp.int32(NB), *([neg] * (D // 16))))
        prev = carry[0]
        acc = carry[1:]
        for f in range(D // 16):
            a = agg_v[prev, pl.ds(f * 16, 16)]
            agg_v[prev, pl.ds(f * 16, 16)] = jnp.maximum(a, acc[f])

    @pl.when(nch > 0)
    def _():
        start(0, rows0, sem0)

    @pl.when(nch > 1)
    def _():
        start(1, rows1, sem1)

    def pair(p, _):
        g0 = 2 * p

        @pl.when(g0 < nch)
        def _():
            wait(rows0, sem0)
            process(g0 * G, rows0)

            @pl.when(g0 + 2 < nch)
            def _():
                start(g0 + 2, rows0, sem0)

        @pl.when(g0 + 1 < nch)
        def _():
            wait(rows1, sem1)
            process((g0 + 1) * G, rows1)

            @pl.when(g0 + 3 < nch)
            def _():
                start(g0 + 3, rows1, sem1)
        return 0

    lax.fori_loop(0, (nch + 1) // 2, pair, 0)

    pltpu.sync_copy(agg_v.at[pl.ds(0, NB)], agg_hbm.at[sc, pl.ds(lo, NB)])


def _sc_agg(xp, ldst, lsrc, cnts):
    mesh = plsc.VectorSubcoreMesh(core_axis_name="c", subcore_axis_name="s")
    f = pl.kernel(
        _sc_agg_body,
        out_type=jax.ShapeDtypeStruct((2, NPAD, D), jnp.float32),
        mesh=mesh,
        scratch_types=[
            pltpu.VMEM((CAP,), jnp.int32),
            pltpu.VMEM((CAP,), jnp.int32),
            pltpu.VMEM((NB + 1, D), jnp.float32),
            pltpu.VMEM((G, D), jnp.float32),
            pltpu.VMEM((G, D), jnp.float32),
            pltpu.VMEM((128,), jnp.int32),
            pltpu.SemaphoreType.DMA,
            pltpu.SemaphoreType.DMA,
        ],
        compiler_params=pltpu.CompilerParams(needs_layout_passes=False),
    )
    return f(xp, ldst, lsrc, cnts)


_BM = 2000


def _agg_specs():
    return [
        pl.BlockSpec((1, _BM, EMB), lambda i: (0, i, 0)),
        pl.BlockSpec((1, _BM, EMB), lambda i: (1, i, 0)),
    ]


def _combine(x, aA, aB):
    a = jnp.maximum(aA[0], aB[0])
    return x + jnp.where(a == _NEG_INF, 0.0, a)


def _mm1_body(x_ref, aa_ref, ab_ref, w_ref, b_ref, oa_ref, ob_ref):
    h = _combine(x_ref[...], aa_ref[...], ab_ref[...])
    y = jnp.dot(h, w_ref[...], preferred_element_type=jnp.float32) + b_ref[...]
    y = jnp.maximum(y, 0.0)
    oa_ref[...] = y[:, :EMB]
    ob_ref[...] = y[:, EMB:]


def _tc_mm1(x, agg1, W1, b1):
    return pl.pallas_call(
        _mm1_body,
        grid=(N // _BM,),
        in_specs=[
            pl.BlockSpec((_BM, EMB), lambda i: (i, 0)),
            *_agg_specs(),
            pl.BlockSpec((EMB, HID), lambda i: (0, 0)),
            pl.BlockSpec((1, HID), lambda i: (0, 0)),
        ],
        out_specs=[
            pl.BlockSpec((_BM, EMB), lambda i: (i, 0)),
            pl.BlockSpec((_BM, EMB), lambda i: (i, 0)),
        ],
        out_shape=[
            jax.ShapeDtypeStruct((N, EMB), jnp.float32),
            jax.ShapeDtypeStruct((N, EMB), jnp.float32),
        ],
    )(x, agg1, agg1, W1, b1.reshape(1, HID))


def _mm2_body(xa_ref, xb_ref, a2aa_ref, a2ab_ref, a2ba_ref, a2bb_ref,
              wa_ref, wb_ref, b_ref, o_ref):
    hL = _combine(xa_ref[...], a2aa_ref[...], a2ab_ref[...])
    hR = _combine(xb_ref[...], a2ba_ref[...], a2bb_ref[...])
    y = jnp.dot(hL, wa_ref[...], preferred_element_type=jnp.float32)
    y += jnp.dot(hR, wb_ref[...], preferred_element_type=jnp.float32)
    o_ref[...] = y + b_ref[...]


def _tc_mm2(x1a, x1b, agg2a, agg2b, W2, b2):
    return pl.pallas_call(
        _mm2_body,
        grid=(N // _BM,),
        in_specs=[
            pl.BlockSpec((_BM, EMB), lambda i: (i, 0)),
            pl.BlockSpec((_BM, EMB), lambda i: (i, 0)),
            *_agg_specs(),
            *_agg_specs(),
            pl.BlockSpec((EMB, EMB), lambda i: (0, 0)),
            pl.BlockSpec((EMB, EMB), lambda i: (0, 0)),
            pl.BlockSpec((1, EMB), lambda i: (0, 0)),
        ],
        out_specs=pl.BlockSpec((_BM, EMB), lambda i: (i, 0)),
        out_shape=jax.ShapeDtypeStruct((N, EMB), jnp.float32),
    )(x1a, x1b, agg2a, agg2a, agg2b, agg2b, W2[:EMB], W2[EMB:], b2.reshape(1, EMB))


def kernel(edge_index, emb_table, W1, b1, W2, b2):
    src = edge_index[0].astype(jnp.int32)
    dst = edge_index[1].astype(jnp.int32)
    ldst, lsrc, cnts = _sc_bin(src, dst)
    agg1 = _sc_agg(emb_table, ldst, lsrc, cnts)
    x1a, x1b = _tc_mm1(emb_table, agg1, W1, b1)
    agg2a = _sc_agg(x1a, ldst, lsrc, cnts)
    agg2b = _sc_agg(x1b, ldst, lsrc, cnts)
    return _tc_mm2(x1a, x1b, agg2a, agg2b, W2, b2)
